# baseline (speedup 1.0000x reference)
import functools

import jax
import jax.numpy as jnp
from jax import lax
from jax.experimental import pallas as pl
from jax.experimental.pallas import tpu as pltpu
from jax.experimental.pallas import tpu_sc as plsc

_BATCH = 16384
_NF = 26
_D = 32
_B = _BATCH * _NF
_V = 1000000
_Q = _V // 4

_info = plsc.get_sparse_core_info()
_NC, _NS = _info.num_cores, _info.num_subcores
_NW = _NC * _NS
_BPW = _B // _NW
_CH = 832
_NCHUNK = _BPW // _CH
_NBUF = 4

_CBLK = 4096
_GRID = -(-_V // _CBLK)
_QROWS = _GRID * (_CBLK // 4)


def _prep_body(w, o):
    xb = w[...]
    eye = jnp.eye(_D, dtype=jnp.float32)
    o[...] = jnp.concatenate(
        [lax.dot_general(
            xb[:, 1024 * kp:1024 * (kp + 1)], eye,
            (((0,), (0,)), ((), ())),
            preferred_element_type=jnp.float32,
            precision=lax.Precision.HIGHEST)
         for kp in range(4)], axis=1)


_prep = pl.pallas_call(
    _prep_body,
    grid=(_GRID,),
    in_specs=[pl.BlockSpec((_D, _CBLK), lambda i: (0, i))],
    out_specs=pl.BlockSpec((_CBLK // 4, 128), lambda i: (i, 0)),
    out_shape=jax.ShapeDtypeStruct((_QROWS, 128), jnp.float32),
)

_mesh = plsc.VectorSubcoreMesh(core_axis_name="c", subcore_axis_name="s")


@functools.partial(
    pl.kernel,
    mesh=_mesh,
    out_type=jax.ShapeDtypeStruct((_B, _D), jnp.float32),
    scratch_types=[
        pltpu.VMEM((_BPW,), jnp.int32),
        pltpu.VMEM((_NBUF * _CH, _D), jnp.float32),
        pltpu.SemaphoreType.DMA,
        pltpu.SemaphoreType.DMA,
    ],
    compiler_params=pltpu.CompilerParams(
        use_tc_tiling_on_sc=False, needs_layout_passes=False),
)
def _emb_lookup(x_hbm, w_hbm, out_hbm, idx_v, rows_v, gsem, osem):
    wid = lax.axis_index("s") * _NC + lax.axis_index("c")
    base = wid * _BPW
    pltpu.sync_copy(x_hbm.at[pl.ds(base, _BPW)], idx_v)

    def remap(t, carry):
        v = idx_v[pl.ds(t * 16, 16)]
        idx_v[pl.ds(t * 16, 16)] = ((v & -4096)
                                    + ((v & 1023) << 2)
                                    + ((v >> 10) & 3))
        return carry

    lax.fori_loop(0, _BPW // 16, remap, 0)

    def gather(j):
        return pltpu.async_copy(
            w_hbm.at[idx_v.at[pl.ds(j * _CH, _CH)]],
            rows_v.at[pl.ds((j % _NBUF) * _CH, _CH)], gsem)

    gathers = [None] * _NCHUNK
    outs = [None] * _NCHUNK
    for j in range(min(_NBUF - 1, _NCHUNK)):
        gathers[j] = gather(j)
    for i in range(_NCHUNK):
        gathers[i].wait()
        outs[i] = pltpu.async_copy(
            rows_v.at[pl.ds((i % _NBUF) * _CH, _CH)],
            out_hbm.at[pl.ds(base + i * _CH, _CH)], osem)
        j = i + _NBUF - 1
        if j < _NCHUNK:
            if j - _NBUF >= 0:
                outs[j - _NBUF].wait()
            gathers[j] = gather(j)
    for i in range(max(0, _NCHUNK - _NBUF), _NCHUNK):
        outs[i].wait()


def kernel(x, weight):
    wq = _prep(weight.T)
    wlin = wq.reshape(_QROWS * 4, _D)
    xf = x.reshape(_B).astype(jnp.int32)
    out = _emb_lookup(xf, wlin)
    return out.reshape(_BATCH, _NF, _D)


# speedup vs baseline: 1.4231x; 1.4231x over previous
import functools

import jax
import jax.numpy as jnp
from jax import lax
from jax.experimental import pallas as pl
from jax.experimental.pallas import tpu as pltpu
from jax.experimental.pallas import tpu_sc as plsc

_BATCH = 16384
_NF = 26
_D = 32
_B = _BATCH * _NF
_V = 1000000
_Q = _V // 4

_info = plsc.get_sparse_core_info()
_NC, _NS = _info.num_cores, _info.num_subcores
_NW = _NC * _NS
_BPW = _B // _NW
_CH = 832
_NCHUNK = _BPW // _CH
_NBUF = 4

_CBLK = 4096
_GRID = -(-_V // _CBLK)
_QROWS = _GRID * (_CBLK // 4)


def _prep_body(w, o):
    xb = w[...]
    o[...] = jnp.concatenate(
        [xb[:, 1024 * kp:1024 * (kp + 1)].T for kp in range(4)], axis=1)


_prep = pl.pallas_call(
    _prep_body,
    grid=(_GRID,),
    in_specs=[pl.BlockSpec((_D, _CBLK), lambda i: (0, i))],
    out_specs=pl.BlockSpec((_CBLK // 4, 128), lambda i: (i, 0)),
    out_shape=jax.ShapeDtypeStruct((_QROWS, 128), jnp.float32),
)

_mesh = plsc.VectorSubcoreMesh(core_axis_name="c", subcore_axis_name="s")


@functools.partial(
    pl.kernel,
    mesh=_mesh,
    out_type=jax.ShapeDtypeStruct((_B, _D), jnp.float32),
    scratch_types=[
        pltpu.VMEM((_BPW,), jnp.int32),
        pltpu.VMEM((_NBUF * _CH, _D), jnp.float32),
        pltpu.SemaphoreType.DMA,
        pltpu.SemaphoreType.DMA,
    ],
    compiler_params=pltpu.CompilerParams(
        use_tc_tiling_on_sc=False, needs_layout_passes=False),
)
def _emb_lookup(x_hbm, w_hbm, out_hbm, idx_v, rows_v, gsem, osem):
    wid = lax.axis_index("s") * _NC + lax.axis_index("c")
    base = wid * _BPW
    pltpu.sync_copy(x_hbm.at[pl.ds(base, _BPW)], idx_v)

    def remap(t, carry):
        v = idx_v[pl.ds(t * 16, 16)]
        idx_v[pl.ds(t * 16, 16)] = ((v & -4096)
                                    + ((v & 1023) << 2)
                                    + ((v >> 10) & 3))
        return carry

    lax.fori_loop(0, _BPW // 16, remap, 0)

    def gather(j):
        return pltpu.async_copy(
            w_hbm.at[idx_v.at[pl.ds(j * _CH, _CH)]],
            rows_v.at[pl.ds((j % _NBUF) * _CH, _CH)], gsem)

    gathers = [None] * _NCHUNK
    outs = [None] * _NCHUNK
    for j in range(min(_NBUF - 1, _NCHUNK)):
        gathers[j] = gather(j)
    for i in range(_NCHUNK):
        gathers[i].wait()
        outs[i] = pltpu.async_copy(
            rows_v.at[pl.ds((i % _NBUF) * _CH, _CH)],
            out_hbm.at[pl.ds(base + i * _CH, _CH)], osem)
        j = i + _NBUF - 1
        if j < _NCHUNK:
            if j - _NBUF >= 0:
                outs[j - _NBUF].wait()
            gathers[j] = gather(j)
    for i in range(max(0, _NCHUNK - _NBUF), _NCHUNK):
        outs[i].wait()


def kernel(x, weight):
    wq = _prep(weight.T)
    wlin = wq.reshape(_QROWS * 4, _D)
    xf = x.reshape(_B).astype(jnp.int32)
    out = _emb_lookup(xf, wlin)
    return out.reshape(_BATCH, _NF, _D)
